# baseline (speedup 1.0000x reference)
import functools

import jax
import jax.numpy as jnp
from jax import lax
from jax.experimental import pallas as pl
from jax.experimental.pallas import tpu as pltpu
from jax.experimental.pallas import tpu_sc as plsc

HID = 128
ATT = 5
NVOC = 475
NNODE = 10000
NQ = 16
NE = 160000
NL = 2
NXL = 4

NC = 2
NS = 16
NW = NC * NS
CH = 128
K = 40
NEP = NW * K * CH
DM = 144
DMH = 80
DOV = 64
NSEG = 10016
BE = 1280
BN = 2000
NKEY = NVOC * NQ
KEYP = 7680


@functools.cache
def _mesh():
    return plsc.VectorSubcoreMesh(core_axis_name="c", subcore_axis_name="s",
                                  num_cores=NC, num_subcores=NS)


_SC_PARAMS = pltpu.CompilerParams(use_tc_tiling_on_sc=False)


def _wid():
    return lax.axis_index("s") * NC + lax.axis_index("c")


GG = 4
GS = 2


def _dma_pipeline(nchunks, G, in_cp, out_start, out_wait):
    def fire_in(g, jb):
        for u in range(G):
            in_cp(jb + u, G * g + u, g).start()

    def drain_in(g, jb):
        for u in range(G):
            in_cp(jb + u, G * g + u, g).wait()

    def fire_out(g, jb):
        for u in range(G):
            out_start(jb + u, G * g + u, g)

    def drain_out(g, jb):
        for u in range(G):
            out_wait(jb + u, G * g + u, g)

    nit = nchunks // (2 * G)
    fire_in(0, 0)
    fire_in(1, G)

    def body(i, _):
        ja = 2 * G * i
        jb = ja + G
        drain_in(0, ja)
        fire_out(0, ja)
        drain_in(1, jb)
        fire_out(1, jb)

        @pl.when(i < nit - 1)
        def _():
            drain_out(0, ja)
            fire_in(0, ja + 2 * G)
            drain_out(1, jb)
            fire_in(1, jb + 2 * G)

        @pl.when(i == nit - 1)
        def _():
            drain_out(0, ja)
            drain_out(1, jb)

        return 0

    lax.fori_loop(0, nit, body, 0)


def _gather_multi_body(*args):
    ntab = (len(args) - 6) // 2
    tabs = args[:ntab]
    idx_hbm = args[ntab]
    outs = args[ntab + 1:2 * ntab + 1]
    idxb, rows, gs0, gs1, ss0, ss1 = args[2 * ntab + 1:]
    w = _wid()
    pltpu.sync_copy(idx_hbm.at[w], idxb)
    base = w * (K * CH)
    gsem = (gs0, gs1)
    ssem = (ss0, ss1)

    for tab, out in zip(tabs, outs):
        def gcp(j, slot, g):
            return pltpu.make_async_copy(tab.at[idxb.at[j]], rows.at[slot],
                                         gsem[g])

        def scp(j, slot, g):
            return pltpu.make_async_copy(
                rows.at[slot], out.at[pl.ds(base + j * CH, CH)], ssem[g])

        _dma_pipeline(K, GG, gcp,
                      lambda j, slot, g: scp(j, slot, g).start(),
                      lambda j, slot, g: scp(j, slot, g).wait())


@jax.jit
def _sc_gather_multi(tabs, idx):
    dt = tabs[0].dtype
    f = pl.kernel(
        _gather_multi_body,
        out_type=[jax.ShapeDtypeStruct((NEP, HID), dt) for _ in tabs],
        mesh=_mesh(),
        scratch_types=[pltpu.VMEM((K, CH), jnp.int32),
                       pltpu.VMEM((2 * GG, CH, HID), dt),
                       pltpu.SemaphoreType.DMA,
                       pltpu.SemaphoreType.DMA,
                       pltpu.SemaphoreType.DMA,
                       pltpu.SemaphoreType.DMA],
        compiler_params=_SC_PARAMS,
    )
    return f(*tabs, idx)


def _sc_gather(tab, idx):
    return _sc_gather_multi((tab,), idx)[0]


def _scatter_body(msgx, obj_idx, zrows, out, idxb, rows, ls0, ls1, as0, as1,
                  shared):
    c = lax.axis_index("c")
    s = lax.axis_index("s")
    seg_per_tile = NSEG // NS
    t0 = s * seg_per_tile
    col0 = c * DOV

    pltpu.sync_copy(zrows, shared.at[pl.ds(t0, seg_per_tile)])
    plsc.subcore_barrier()

    def run_span(w):
        pltpu.sync_copy(obj_idx.at[w], idxb)
        base = w * (K * CH)
        lsem = (ls0, ls1)
        asem = (as0, as1)

        def lcp(j, slot, g):
            return pltpu.make_async_copy(
                msgx.at[pl.ds(base + j * CH, CH), pl.ds(col0, DMH)],
                rows.at[slot], lsem[g])

        def astart(j, slot, g):
            pltpu.async_copy(rows.at[slot], shared.at[idxb.at[j]], asem[g],
                             add=True)

        def await_(j, slot, g):
            pltpu.make_async_copy(rows.at[slot], shared.at[idxb.at[j]],
                                  asem[g]).wait()

        _dma_pipeline(K, GS, lcp, astart, await_)

    run_span(2 * s)
    run_span(2 * s + 1)
    plsc.subcore_barrier()
    pltpu.sync_copy(shared.at[pl.ds(t0, seg_per_tile)],
                    out.at[pl.ds(t0, seg_per_tile), pl.ds(col0, DMH)])


@jax.jit
def _sc_scatter(msgx, obj_idx, zrows):
    f = pl.kernel(
        _scatter_body,
        out_type=jax.ShapeDtypeStruct((NSEG, DM), jnp.float32),
        mesh=_mesh(),
        scratch_types=[pltpu.VMEM((K, CH), jnp.int32),
                       pltpu.VMEM((2 * GS, CH, DMH), jnp.float32),
                       pltpu.SemaphoreType.DMA,
                       pltpu.SemaphoreType.DMA,
                       pltpu.SemaphoreType.DMA,
                       pltpu.SemaphoreType.DMA,
                       pltpu.VMEM_SHARED((NSEG, DMH), jnp.float32)],
        compiler_params=_SC_PARAMS,
    )
    return f(msgx, obj_idx, zrows)


def _l0_body(m0s, key_idx, obj_idx, zrows, out, keyb, objb, rows,
             ls0, ls1, as0, as1, shared):
    c = lax.axis_index("c")
    s = lax.axis_index("s")
    seg_per_tile = NSEG // NS
    t0 = s * seg_per_tile

    pltpu.sync_copy(zrows, shared.at[pl.ds(t0, seg_per_tile)])
    plsc.subcore_barrier()

    m0c = m0s.at[c]

    def run_span(w):
        pltpu.sync_copy(key_idx.at[w], keyb)
        pltpu.sync_copy(obj_idx.at[w], objb)
        lsem = (ls0, ls1)
        asem = (as0, as1)

        def lcp(j, slot, g):
            return pltpu.make_async_copy(m0c.at[keyb.at[j]], rows.at[slot],
                                         lsem[g])

        def astart(j, slot, g):
            pltpu.async_copy(rows.at[slot], shared.at[objb.at[j]], asem[g],
                             add=True)

        def await_(j, slot, g):
            pltpu.make_async_copy(rows.at[slot], shared.at[objb.at[j]],
                                  asem[g]).wait()

        _dma_pipeline(K, GS, lcp, astart, await_)

    run_span(2 * s)
    run_span(2 * s + 1)
    plsc.subcore_barrier()
    pltpu.sync_copy(shared.at[pl.ds(t0, seg_per_tile)],
                    out.at[pl.ds(t0, seg_per_tile), pl.ds(c * DOV, DMH)])


@jax.jit
def _sc_l0(m0s, key_idx, obj_idx, zrows):
    f = pl.kernel(
        _l0_body,
        out_type=jax.ShapeDtypeStruct((NSEG, DM), jnp.float32),
        mesh=_mesh(),
        scratch_types=[pltpu.VMEM((K, CH), jnp.int32),
                       pltpu.VMEM((K, CH), jnp.int32),
                       pltpu.VMEM((2 * GS, CH, DMH), jnp.float32),
                       pltpu.SemaphoreType.DMA,
                       pltpu.SemaphoreType.DMA,
                       pltpu.SemaphoreType.DMA,
                       pltpu.SemaphoreType.DMA,
                       pltpu.VMEM_SHARED((NSEG, DMH), jnp.float32)],
        compiler_params=_SC_PARAMS,
    )
    return f(m0s, key_idx, obj_idx, zrows)


def _edge_body(hs_ref, hr_ref, r_ref, qre_ref, g1b_ref, g1m_ref, gb1_ref,
               g2b_ref, gb2_ref, Ws_ref, Wqr_ref, bqr_ref, wa_ref, ba_ref,
               out_ref):
    hsb = hs_ref[...]
    hrb = hr_ref[...]
    hs = hsb.astype(jnp.float32)
    nb = hs.shape[0]
    r = r_ref[0]
    iot = lax.broadcasted_iota(jnp.int32, (NQ, nb), 0)
    oh = (iot == r).astype(jnp.float32)
    qre = qre_ref[...]
    dn = (((0,), (0,)), ((), ()))
    f32 = jnp.float32
    g1b = g1b_ref[...]
    qg1 = qre @ g1m_ref[...]
    g = (lax.dot_general(hrb, g1b[:HID], (((1,), (0,)), ((), ())),
                         preferred_element_type=f32) +
         lax.dot_general(hsb, g1b[2 * HID:], (((1,), (0,)), ((), ())),
                         preferred_element_type=f32) +
         lax.dot_general(oh, qg1, dn, preferred_element_type=f32))
    g = 1.0 / (1.0 + jnp.exp(-(g + gb1_ref[...])))
    u = g[:, :HID]
    rs = g[:, HID:]
    g2b = g2b_ref[...]
    rh = (rs * hs).astype(jnp.bfloat16)
    cand = jnp.tanh(
        lax.dot_general(hrb, g2b[:HID], (((1,), (0,)), ((), ())),
                        preferred_element_type=f32) +
        lax.dot_general(rh, g2b[HID:], (((1,), (0,)), ((), ())),
                        preferred_element_type=f32) + gb2_ref[...])
    uri = (1.0 - u) * hs + u * cand
    qa = qre @ Wqr_ref[...] + bqr_ref[...]
    sc = jnp.maximum(uri @ Ws_ref[...] +
                     lax.dot_general(oh, qa, dn,
                                     preferred_element_type=f32), 0.0)
    aw = sc @ wa_ref[...] + ba_ref[...]
    ue = jnp.exp(aw)
    msg = ue * uri
    out_ref[...] = jnp.concatenate(
        [msg[:, :DOV], ue, jnp.zeros((nb, DM - 2 * DOV - 1), jnp.float32),
         msg[:, DOV:]], axis=1)


def _full(shape):
    nd = len(shape)
    return pl.BlockSpec(shape, lambda i, _n=nd: (0,) * _n)


@jax.jit
def _tc_edge(hs, hr, r3, qre, g1b, g1m, gb1, g2b, gb2, Ws, Wqr, bqr, wa, ba):
    ne = hs.shape[0]
    nblk = ne // BE
    return pl.pallas_call(
        _edge_body,
        out_shape=jax.ShapeDtypeStruct((ne, DM), jnp.float32),
        grid=(nblk,),
        in_specs=[
            pl.BlockSpec((BE, HID), lambda i: (i, 0)),
            pl.BlockSpec((BE, HID), lambda i: (i, 0)),
            pl.BlockSpec((1, 1, BE), lambda i: (i, 0, 0)),
            _full((NQ, HID)), _full((3 * HID, 2 * HID)),
            _full((HID, 2 * HID)), _full((1, 2 * HID)),
            _full((2 * HID, HID)), _full((1, HID)), _full((HID, ATT)),
            _full((HID, ATT)), _full((1, ATT)), _full((ATT, 1)),
            _full((1, 1)),
        ],
        out_specs=pl.BlockSpec((BE, DM), lambda i: (i, 0)),
    )(hs, hr, r3, qre, g1b, g1m, gb1, g2b, gb2, Ws, Wqr, bqr, wa, ba)


def _node_body(a0_ref, ng_ref, h0_ref, qre_ref, Wh_ref, g1b_ref, g1m_ref,
               gb1_ref, g2b_ref, gb2_ref, Wf_ref, hid_ref, hbf_ref, sc_ref):
    a = a0_ref[...]
    agg = jnp.concatenate([a[:, :DOV], a[:, DMH:]], axis=1)
    se = a[:, DOV:DOV + 1]
    m = jnp.maximum((agg / se) @ Wh_ref[...], 0.0)
    ng = ng_ref[0]
    iot = lax.broadcasted_iota(jnp.int32, (NQ, BN), 0)
    oh = (iot == ng).astype(jnp.float32)
    dn = (((0,), (0,)), ((), ()))
    f32 = jnp.float32
    h = h0_ref[...]
    g1 = g1b_ref[...]
    qg1 = qre_ref[...] @ g1m_ref[...]
    g = (m @ g1[:HID] + h @ g1[2 * HID:] +
         lax.dot_general(oh, qg1, dn, preferred_element_type=f32))
    g = 1.0 / (1.0 + jnp.exp(-(g + gb1_ref[...])))
    u = g[:, :HID]
    rs = g[:, HID:]
    g2 = g2b_ref[...]
    cand = jnp.tanh(m @ g2[:HID] + (rs * h) @ g2[HID:] + gb2_ref[...])
    out = (1.0 - u) * h + u * cand
    hid_ref[...] = out
    hbf_ref[...] = out.astype(jnp.bfloat16)
    sc_ref[...] = out @ Wf_ref[...]


@jax.jit
def _tc_node(a0, ng3, h0, qre, Wh, g1b, g1m, gb1, g2b, gb2, Wf):
    nblk = NNODE // BN
    return pl.pallas_call(
        _node_body,
        out_shape=[jax.ShapeDtypeStruct((NNODE, HID), jnp.float32),
                   jax.ShapeDtypeStruct((NNODE, HID), jnp.bfloat16),
                   jax.ShapeDtypeStruct((NNODE, 1), jnp.float32)],
        grid=(nblk,),
        in_specs=[
            pl.BlockSpec((BN, DM), lambda i: (i, 0)),
            pl.BlockSpec((1, 1, BN), lambda i: (i, 0, 0)),
            pl.BlockSpec((BN, HID), lambda i: (i, 0)),
            _full((NQ, HID)), _full((HID, HID)), _full((3 * HID, 2 * HID)),
            _full((HID, 2 * HID)), _full((1, 2 * HID)),
            _full((2 * HID, HID)), _full((1, HID)),
            _full((HID, 1)),
        ],
        out_specs=[pl.BlockSpec((BN, HID), lambda i: (i, 0)),
                   pl.BlockSpec((BN, HID), lambda i: (i, 0)),
                   pl.BlockSpec((BN, 1), lambda i: (i, 0))],
    )(a0, ng3, h0, qre, Wh, g1b, g1m, gb1, g2b, gb2, Wf)


def kernel(subs, rels, edges, nodes, old_idx, params):
    n = subs.shape[0]
    idt = edges.dtype
    r_idx = edges[:, 0]
    rel = edges[:, 2]
    sub = edges[:, 4]
    obj = edges[:, 5]
    sub2 = old_idx[sub]
    key = rel * NQ + r_idx

    node_group = jnp.zeros((NNODE,), dtype=idt).at[obj].set(r_idx)

    pad = NEP - NE

    def p32(x, fill):
        return jnp.concatenate(
            [x, jnp.full((pad,), fill, x.dtype)]).reshape(NW, K, CH)

    sub_sc = p32(sub, 0)
    sub2_sc = p32(sub2, 0)
    rel_sc = p32(rel, 0)
    obj_sc = p32(obj, NNODE)
    key_sc = p32(key, 0)
    r3 = jnp.concatenate([r_idx, jnp.zeros((pad,), idt)]).reshape(
        NEP // BE, 1, BE)
    ng3 = node_group.reshape(NNODE // BN, 1, BN)
    zrows = jnp.zeros((NSEG // NS, DMH), jnp.float32)

    layers = params["layers"]
    top = params["top"]
    bf = jnp.bfloat16
    tp = [top["gW1"], top["gW1"][HID:2 * HID],
          top["gb1"].reshape(1, -1), top["gW2"],
          top["gb2"].reshape(1, -1)]
    Wf = params["Wfinal"]

    def edge_call(p, hs, hr, r3v, qre):
        return _tc_edge(hs, hr, r3v, qre, p["gW1"].astype(bf),
                        p["gW1"][HID:2 * HID], p["gb1"].reshape(1, -1),
                        p["gW2"].astype(bf), p["gb2"].reshape(1, -1),
                        p["Ws"], p["Wqr"], p["bqr"].reshape(1, -1), p["wa"],
                        p["ba"].reshape(1, -1))

    def node_call(p, aggx, h0_in, qre):
        return _tc_node(aggx[:NNODE], ng3, h0_in, qre, p["Wh"],
                        tp[0], tp[1], tp[2], tp[3], tp[4], Wf)

    qres = [p["rela"][rels] for p in layers]

    hr_tabs = [layers[li]["rela"].astype(bf) for li in range(1, NL + NXL)]
    hr_outs = _sc_gather_multi(tuple(hr_tabs), rel_sc)
    hr_l = [None] + list(hr_outs)

    p0 = layers[0]
    hs0 = jnp.zeros((KEYP, HID), jnp.bfloat16)
    hr0 = jnp.concatenate(
        [jnp.repeat(p0["rela"], NQ, axis=0),
         jnp.zeros((KEYP - NKEY, HID), jnp.float32)]).astype(jnp.bfloat16)
    r0 = jnp.tile(jnp.arange(NQ, dtype=idt), KEYP // NQ).reshape(
        KEYP // BE, 1, BE)
    m0 = edge_call(p0, hs0, hr0, r0, qres[0])
    m0s = jnp.stack([m0[:, :DMH], m0[:, DOV:]])
    aggx = _sc_l0(m0s, key_sc, obj_sc, zrows)
    zeros_h = jnp.zeros((NNODE, HID), jnp.float32)
    hidden, hidden_bf, _ = node_call(p0, aggx, zeros_h, qres[0])
    h0 = hidden
    h0_in = jnp.zeros((NNODE, HID), jnp.float32).at[old_idx].set(h0)

    for li in range(1, NL + NXL):
        p = layers[li]
        sub_idx = sub_sc if li < NL else sub2_sc
        hs = _sc_gather(hidden_bf, sub_idx)
        msgx = edge_call(p, hs, hr_l[li], r3, qres[li])
        aggx = _sc_scatter(msgx, obj_sc, zrows)
        hidden, hidden_bf, sc = node_call(p, aggx, h0_in, qres[li])
        h0_in = hidden

    scores = sc[:, 0]
    scores_all = jnp.zeros((n, NNODE), jnp.float32).at[
        nodes[:, 0], nodes[:, 1]].set(scores)
    return scores_all


# speedup vs baseline: 1.0022x; 1.0022x over previous
import functools

import jax
import jax.numpy as jnp
from jax import lax
from jax.experimental import pallas as pl
from jax.experimental.pallas import tpu as pltpu
from jax.experimental.pallas import tpu_sc as plsc

HID = 128
ATT = 5
NVOC = 475
NNODE = 10000
NQ = 16
NE = 160000
NL = 2
NXL = 4

NC = 2
NS = 16
NW = NC * NS
CH = 128
K = 40
NEP = NW * K * CH
DM = 144
DMH = 80
DOV = 64
NSEG = 10016
BE = 1280
BN = 2000
NKEY = NVOC * NQ
KEYP = 7680


@functools.cache
def _mesh():
    return plsc.VectorSubcoreMesh(core_axis_name="c", subcore_axis_name="s",
                                  num_cores=NC, num_subcores=NS)


_SC_PARAMS = pltpu.CompilerParams(use_tc_tiling_on_sc=False)


def _wid():
    return lax.axis_index("s") * NC + lax.axis_index("c")


GG = 2
GS = 2


def _dma_pipeline(nchunks, G, in_cp, out_start, out_wait):
    def fire_in(g, jb):
        for u in range(G):
            in_cp(jb + u, G * g + u, g).start()

    def drain_in(g, jb):
        for u in range(G):
            in_cp(jb + u, G * g + u, g).wait()

    def fire_out(g, jb):
        for u in range(G):
            out_start(jb + u, G * g + u, g)

    def drain_out(g, jb):
        for u in range(G):
            out_wait(jb + u, G * g + u, g)

    nit = nchunks // (2 * G)
    fire_in(0, 0)
    fire_in(1, G)

    def body(i, _):
        ja = 2 * G * i
        jb = ja + G
        drain_in(0, ja)
        fire_out(0, ja)
        drain_in(1, jb)
        fire_out(1, jb)

        @pl.when(i < nit - 1)
        def _():
            drain_out(0, ja)
            fire_in(0, ja + 2 * G)
            drain_out(1, jb)
            fire_in(1, jb + 2 * G)

        @pl.when(i == nit - 1)
        def _():
            drain_out(0, ja)
            drain_out(1, jb)

        return 0

    lax.fori_loop(0, nit, body, 0)


def _gather_multi_body(*args):
    ntab = (len(args) - 6) // 2
    tabs = args[:ntab]
    idx_hbm = args[ntab]
    outs = args[ntab + 1:2 * ntab + 1]
    idxb, rows, gs0, gs1, ss0, ss1 = args[2 * ntab + 1:]
    w = _wid()
    pltpu.sync_copy(idx_hbm.at[w], idxb)
    base = w * (K * CH)
    gsem = (gs0, gs1)
    ssem = (ss0, ss1)

    for tab, out in zip(tabs, outs):
        def gcp(j, slot, g):
            return pltpu.make_async_copy(tab.at[idxb.at[j]], rows.at[slot],
                                         gsem[g])

        def scp(j, slot, g):
            return pltpu.make_async_copy(
                rows.at[slot], out.at[pl.ds(base + j * CH, CH)], ssem[g])

        _dma_pipeline(K, GG, gcp,
                      lambda j, slot, g: scp(j, slot, g).start(),
                      lambda j, slot, g: scp(j, slot, g).wait())


@jax.jit
def _sc_gather_multi(tabs, idx):
    dt = tabs[0].dtype
    f = pl.kernel(
        _gather_multi_body,
        out_type=[jax.ShapeDtypeStruct((NEP, HID), dt) for _ in tabs],
        mesh=_mesh(),
        scratch_types=[pltpu.VMEM((K, CH), jnp.int32),
                       pltpu.VMEM((2 * GG, CH, HID), dt),
                       pltpu.SemaphoreType.DMA,
                       pltpu.SemaphoreType.DMA,
                       pltpu.SemaphoreType.DMA,
                       pltpu.SemaphoreType.DMA],
        compiler_params=_SC_PARAMS,
    )
    return f(*tabs, idx)


def _sc_gather(tab, idx):
    return _sc_gather_multi((tab,), idx)[0]


def _scatter_body(msgx, obj_idx, zrows, out, idxb, rows, ls0, ls1, as0, as1,
                  shared):
    c = lax.axis_index("c")
    s = lax.axis_index("s")
    seg_per_tile = NSEG // NS
    t0 = s * seg_per_tile
    col0 = c * DOV

    pltpu.sync_copy(zrows, shared.at[pl.ds(t0, seg_per_tile)])
    plsc.subcore_barrier()

    def run_span(w):
        pltpu.sync_copy(obj_idx.at[w], idxb)
        base = w * (K * CH)
        lsem = (ls0, ls1)
        asem = (as0, as1)

        def lcp(j, slot, g):
            return pltpu.make_async_copy(
                msgx.at[pl.ds(base + j * CH, CH), pl.ds(col0, DMH)],
                rows.at[slot], lsem[g])

        def astart(j, slot, g):
            pltpu.async_copy(rows.at[slot], shared.at[idxb.at[j]], asem[g],
                             add=True)

        def await_(j, slot, g):
            pltpu.make_async_copy(rows.at[slot], shared.at[idxb.at[j]],
                                  asem[g]).wait()

        _dma_pipeline(K, GS, lcp, astart, await_)

    run_span(2 * s)
    run_span(2 * s + 1)
    plsc.subcore_barrier()
    pltpu.sync_copy(shared.at[pl.ds(t0, seg_per_tile)],
                    out.at[pl.ds(t0, seg_per_tile), pl.ds(col0, DMH)])


@jax.jit
def _sc_scatter(msgx, obj_idx, zrows):
    f = pl.kernel(
        _scatter_body,
        out_type=jax.ShapeDtypeStruct((NSEG, DM), jnp.float32),
        mesh=_mesh(),
        scratch_types=[pltpu.VMEM((K, CH), jnp.int32),
                       pltpu.VMEM((2 * GS, CH, DMH), jnp.float32),
                       pltpu.SemaphoreType.DMA,
                       pltpu.SemaphoreType.DMA,
                       pltpu.SemaphoreType.DMA,
                       pltpu.SemaphoreType.DMA,
                       pltpu.VMEM_SHARED((NSEG, DMH), jnp.float32)],
        compiler_params=_SC_PARAMS,
    )
    return f(msgx, obj_idx, zrows)


def _l0_body(m0s, key_idx, obj_idx, zrows, out, keyb, objb, rows,
             ls0, ls1, as0, as1, shared):
    c = lax.axis_index("c")
    s = lax.axis_index("s")
    seg_per_tile = NSEG // NS
    t0 = s * seg_per_tile

    pltpu.sync_copy(zrows, shared.at[pl.ds(t0, seg_per_tile)])
    plsc.subcore_barrier()

    m0c = m0s.at[c]

    def run_span(w):
        pltpu.sync_copy(key_idx.at[w], keyb)
        pltpu.sync_copy(obj_idx.at[w], objb)
        lsem = (ls0, ls1)
        asem = (as0, as1)

        def lcp(j, slot, g):
            return pltpu.make_async_copy(m0c.at[keyb.at[j]], rows.at[slot],
                                         lsem[g])

        def astart(j, slot, g):
            pltpu.async_copy(rows.at[slot], shared.at[objb.at[j]], asem[g],
                             add=True)

        def await_(j, slot, g):
            pltpu.make_async_copy(rows.at[slot], shared.at[objb.at[j]],
                                  asem[g]).wait()

        _dma_pipeline(K, GS, lcp, astart, await_)

    run_span(2 * s)
    run_span(2 * s + 1)
    plsc.subcore_barrier()
    pltpu.sync_copy(shared.at[pl.ds(t0, seg_per_tile)],
                    out.at[pl.ds(t0, seg_per_tile), pl.ds(c * DOV, DMH)])


@jax.jit
def _sc_l0(m0s, key_idx, obj_idx, zrows):
    f = pl.kernel(
        _l0_body,
        out_type=jax.ShapeDtypeStruct((NSEG, DM), jnp.float32),
        mesh=_mesh(),
        scratch_types=[pltpu.VMEM((K, CH), jnp.int32),
                       pltpu.VMEM((K, CH), jnp.int32),
                       pltpu.VMEM((2 * GS, CH, DMH), jnp.float32),
                       pltpu.SemaphoreType.DMA,
                       pltpu.SemaphoreType.DMA,
                       pltpu.SemaphoreType.DMA,
                       pltpu.SemaphoreType.DMA,
                       pltpu.VMEM_SHARED((NSEG, DMH), jnp.float32)],
        compiler_params=_SC_PARAMS,
    )
    return f(m0s, key_idx, obj_idx, zrows)


def _edge_body(hs_ref, hr_ref, r_ref, qre_ref, g1b_ref, g1m_ref, gb1_ref,
               g2b_ref, gb2_ref, Ws_ref, Wqr_ref, bqr_ref, wa_ref, ba_ref,
               out_ref):
    hsb = hs_ref[...]
    hrb = hr_ref[...]
    hs = hsb.astype(jnp.float32)
    nb = hs.shape[0]
    r = r_ref[0]
    iot = lax.broadcasted_iota(jnp.int32, (NQ, nb), 0)
    oh = (iot == r).astype(jnp.float32)
    qre = qre_ref[...]
    dn = (((0,), (0,)), ((), ()))
    f32 = jnp.float32
    g1b = g1b_ref[...]
    qg1 = qre @ g1m_ref[...]
    g = (lax.dot_general(hrb, g1b[:HID], (((1,), (0,)), ((), ())),
                         preferred_element_type=f32) +
         lax.dot_general(hsb, g1b[2 * HID:], (((1,), (0,)), ((), ())),
                         preferred_element_type=f32) +
         lax.dot_general(oh, qg1, dn, preferred_element_type=f32))
    g = 1.0 / (1.0 + jnp.exp(-(g + gb1_ref[...])))
    u = g[:, :HID]
    rs = g[:, HID:]
    g2b = g2b_ref[...]
    rh = (rs * hs).astype(jnp.bfloat16)
    cand = jnp.tanh(
        lax.dot_general(hrb, g2b[:HID], (((1,), (0,)), ((), ())),
                        preferred_element_type=f32) +
        lax.dot_general(rh, g2b[HID:], (((1,), (0,)), ((), ())),
                        preferred_element_type=f32) + gb2_ref[...])
    uri = (1.0 - u) * hs + u * cand
    qa = qre @ Wqr_ref[...] + bqr_ref[...]
    sc = jnp.maximum(uri @ Ws_ref[...] +
                     lax.dot_general(oh, qa, dn,
                                     preferred_element_type=f32), 0.0)
    aw = sc @ wa_ref[...] + ba_ref[...]
    ue = jnp.exp(aw)
    msg = ue * uri
    out_ref[...] = jnp.concatenate(
        [msg[:, :DOV], ue, jnp.zeros((nb, DM - 2 * DOV - 1), jnp.float32),
         msg[:, DOV:]], axis=1)


def _full(shape):
    nd = len(shape)
    return pl.BlockSpec(shape, lambda i, _n=nd: (0,) * _n)


@jax.jit
def _tc_edge(hs, hr, r3, qre, g1b, g1m, gb1, g2b, gb2, Ws, Wqr, bqr, wa, ba):
    ne = hs.shape[0]
    nblk = ne // BE
    return pl.pallas_call(
        _edge_body,
        out_shape=jax.ShapeDtypeStruct((ne, DM), jnp.float32),
        grid=(nblk,),
        in_specs=[
            pl.BlockSpec((BE, HID), lambda i: (i, 0)),
            pl.BlockSpec((BE, HID), lambda i: (i, 0)),
            pl.BlockSpec((1, 1, BE), lambda i: (i, 0, 0)),
            _full((NQ, HID)), _full((3 * HID, 2 * HID)),
            _full((HID, 2 * HID)), _full((1, 2 * HID)),
            _full((2 * HID, HID)), _full((1, HID)), _full((HID, ATT)),
            _full((HID, ATT)), _full((1, ATT)), _full((ATT, 1)),
            _full((1, 1)),
        ],
        out_specs=pl.BlockSpec((BE, DM), lambda i: (i, 0)),
    )(hs, hr, r3, qre, g1b, g1m, gb1, g2b, gb2, Ws, Wqr, bqr, wa, ba)


def _node_body(a0_ref, ng_ref, h0_ref, qre_ref, Wh_ref, g1b_ref, g1m_ref,
               gb1_ref, g2b_ref, gb2_ref, Wf_ref, hid_ref, hbf_ref, sc_ref):
    a = a0_ref[...]
    agg = jnp.concatenate([a[:, :DOV], a[:, DMH:]], axis=1)
    se = a[:, DOV:DOV + 1]
    m = jnp.maximum((agg / se) @ Wh_ref[...], 0.0)
    ng = ng_ref[0]
    iot = lax.broadcasted_iota(jnp.int32, (NQ, BN), 0)
    oh = (iot == ng).astype(jnp.float32)
    dn = (((0,), (0,)), ((), ()))
    f32 = jnp.float32
    h = h0_ref[...]
    g1 = g1b_ref[...]
    qg1 = qre_ref[...] @ g1m_ref[...]
    g = (m @ g1[:HID] + h @ g1[2 * HID:] +
         lax.dot_general(oh, qg1, dn, preferred_element_type=f32))
    g = 1.0 / (1.0 + jnp.exp(-(g + gb1_ref[...])))
    u = g[:, :HID]
    rs = g[:, HID:]
    g2 = g2b_ref[...]
    cand = jnp.tanh(m @ g2[:HID] + (rs * h) @ g2[HID:] + gb2_ref[...])
    out = (1.0 - u) * h + u * cand
    hid_ref[...] = out
    hbf_ref[...] = out.astype(jnp.bfloat16)
    sc_ref[...] = out @ Wf_ref[...]


@jax.jit
def _tc_node(a0, ng3, h0, qre, Wh, g1b, g1m, gb1, g2b, gb2, Wf):
    nblk = NNODE // BN
    return pl.pallas_call(
        _node_body,
        out_shape=[jax.ShapeDtypeStruct((NNODE, HID), jnp.float32),
                   jax.ShapeDtypeStruct((NNODE, HID), jnp.bfloat16),
                   jax.ShapeDtypeStruct((NNODE, 1), jnp.float32)],
        grid=(nblk,),
        in_specs=[
            pl.BlockSpec((BN, DM), lambda i: (i, 0)),
            pl.BlockSpec((1, 1, BN), lambda i: (i, 0, 0)),
            pl.BlockSpec((BN, HID), lambda i: (i, 0)),
            _full((NQ, HID)), _full((HID, HID)), _full((3 * HID, 2 * HID)),
            _full((HID, 2 * HID)), _full((1, 2 * HID)),
            _full((2 * HID, HID)), _full((1, HID)),
            _full((HID, 1)),
        ],
        out_specs=[pl.BlockSpec((BN, HID), lambda i: (i, 0)),
                   pl.BlockSpec((BN, HID), lambda i: (i, 0)),
                   pl.BlockSpec((BN, 1), lambda i: (i, 0))],
    )(a0, ng3, h0, qre, Wh, g1b, g1m, gb1, g2b, gb2, Wf)


def kernel(subs, rels, edges, nodes, old_idx, params):
    n = subs.shape[0]
    idt = edges.dtype
    r_idx = edges[:, 0]
    rel = edges[:, 2]
    sub = edges[:, 4]
    obj = edges[:, 5]
    sub2 = old_idx[sub]
    key = rel * NQ + r_idx

    node_group = jnp.zeros((NNODE,), dtype=idt).at[obj].set(r_idx)

    pad = NEP - NE

    def p32(x, fill):
        return jnp.concatenate(
            [x, jnp.full((pad,), fill, x.dtype)]).reshape(NW, K, CH)

    sub_sc = p32(sub, 0)
    sub2_sc = p32(sub2, 0)
    rel_sc = p32(rel, 0)
    obj_sc = p32(obj, NNODE)
    key_sc = p32(key, 0)
    r3 = jnp.concatenate([r_idx, jnp.zeros((pad,), idt)]).reshape(
        NEP // BE, 1, BE)
    ng3 = node_group.reshape(NNODE // BN, 1, BN)
    zrows = jnp.zeros((NSEG // NS, DMH), jnp.float32)

    layers = params["layers"]
    top = params["top"]
    bf = jnp.bfloat16
    tp = [top["gW1"], top["gW1"][HID:2 * HID],
          top["gb1"].reshape(1, -1), top["gW2"],
          top["gb2"].reshape(1, -1)]
    Wf = params["Wfinal"]

    def edge_call(p, hs, hr, r3v, qre):
        return _tc_edge(hs, hr, r3v, qre, p["gW1"].astype(bf),
                        p["gW1"][HID:2 * HID], p["gb1"].reshape(1, -1),
                        p["gW2"].astype(bf), p["gb2"].reshape(1, -1),
                        p["Ws"], p["Wqr"], p["bqr"].reshape(1, -1), p["wa"],
                        p["ba"].reshape(1, -1))

    def node_call(p, aggx, h0_in, qre):
        return _tc_node(aggx[:NNODE], ng3, h0_in, qre, p["Wh"],
                        tp[0], tp[1], tp[2], tp[3], tp[4], Wf)

    qres = [p["rela"][rels] for p in layers]

    hr_tabs = [layers[li]["rela"].astype(bf) for li in range(1, NL + NXL)]
    hr_outs = _sc_gather_multi(tuple(hr_tabs), rel_sc)
    hr_l = [None] + list(hr_outs)

    p0 = layers[0]
    hs0 = jnp.zeros((KEYP, HID), jnp.bfloat16)
    hr0 = jnp.concatenate(
        [jnp.repeat(p0["rela"], NQ, axis=0),
         jnp.zeros((KEYP - NKEY, HID), jnp.float32)]).astype(jnp.bfloat16)
    r0 = jnp.tile(jnp.arange(NQ, dtype=idt), KEYP // NQ).reshape(
        KEYP // BE, 1, BE)
    m0 = edge_call(p0, hs0, hr0, r0, qres[0])
    m0s = jnp.stack([m0[:, :DMH], m0[:, DOV:]])
    aggx = _sc_l0(m0s, key_sc, obj_sc, zrows)
    zeros_h = jnp.zeros((NNODE, HID), jnp.float32)
    hidden, hidden_bf, _ = node_call(p0, aggx, zeros_h, qres[0])
    h0 = hidden
    h0_in = jnp.zeros((NNODE, HID), jnp.float32).at[old_idx].set(h0)

    for li in range(1, NL + NXL):
        p = layers[li]
        sub_idx = sub_sc if li < NL else sub2_sc
        hs = _sc_gather(hidden_bf, sub_idx)
        msgx = edge_call(p, hs, hr_l[li], r3, qres[li])
        aggx = _sc_scatter(msgx, obj_sc, zrows)
        hidden, hidden_bf, sc = node_call(p, aggx, h0_in, qres[li])
        h0_in = hidden

    scores = sc[:, 0]
    scores_all = jnp.zeros((n, NNODE), jnp.float32).at[
        nodes[:, 0], nodes[:, 1]].set(scores)
    return scores_all


# speedup vs baseline: 1.0283x; 1.0260x over previous
import functools

import jax
import jax.numpy as jnp
from jax import lax
from jax.experimental import pallas as pl
from jax.experimental.pallas import tpu as pltpu
from jax.experimental.pallas import tpu_sc as plsc

HID = 128
ATT = 5
NVOC = 475
NNODE = 10000
NQ = 16
NE = 160000
NL = 2
NXL = 4

NC = 2
NS = 16
NW = NC * NS
CH = 128
K = 40
NEP = NW * K * CH
DM = 160
DMH = 80
DOV = 80
NSEG = 10016
BE = 1280
BN = 2000
NKEY = NVOC * NQ
KEYP = 7680


@functools.cache
def _mesh():
    return plsc.VectorSubcoreMesh(core_axis_name="c", subcore_axis_name="s",
                                  num_cores=NC, num_subcores=NS)


_SC_PARAMS = pltpu.CompilerParams(use_tc_tiling_on_sc=False)


def _wid():
    return lax.axis_index("s") * NC + lax.axis_index("c")


GG = 2
GS = 2


def _dma_pipeline(nchunks, G, in_cp, out_start, out_wait):
    def fire_in(g, jb):
        for u in range(G):
            in_cp(jb + u, G * g + u, g).start()

    def drain_in(g, jb):
        for u in range(G):
            in_cp(jb + u, G * g + u, g).wait()

    def fire_out(g, jb):
        for u in range(G):
            out_start(jb + u, G * g + u, g)

    def drain_out(g, jb):
        for u in range(G):
            out_wait(jb + u, G * g + u, g)

    nit = nchunks // (2 * G)
    fire_in(0, 0)
    fire_in(1, G)

    def body(i, _):
        ja = 2 * G * i
        jb = ja + G
        drain_in(0, ja)
        fire_out(0, ja)
        drain_in(1, jb)
        fire_out(1, jb)

        @pl.when(i < nit - 1)
        def _():
            drain_out(0, ja)
            fire_in(0, ja + 2 * G)
            drain_out(1, jb)
            fire_in(1, jb + 2 * G)

        @pl.when(i == nit - 1)
        def _():
            drain_out(0, ja)
            drain_out(1, jb)

        return 0

    lax.fori_loop(0, nit, body, 0)


def _gather_multi_body(*args):
    ntab = (len(args) - 6) // 2
    tabs = args[:ntab]
    idx_hbm = args[ntab]
    outs = args[ntab + 1:2 * ntab + 1]
    idxb, rows, gs0, gs1, ss0, ss1 = args[2 * ntab + 1:]
    w = _wid()
    pltpu.sync_copy(idx_hbm.at[w], idxb)
    base = w * (K * CH)
    gsem = (gs0, gs1)
    ssem = (ss0, ss1)

    for tab, out in zip(tabs, outs):
        def gcp(j, slot, g):
            return pltpu.make_async_copy(tab.at[idxb.at[j]], rows.at[slot],
                                         gsem[g])

        def scp(j, slot, g):
            return pltpu.make_async_copy(
                rows.at[slot], out.at[pl.ds(base + j * CH, CH)], ssem[g])

        _dma_pipeline(K, GG, gcp,
                      lambda j, slot, g: scp(j, slot, g).start(),
                      lambda j, slot, g: scp(j, slot, g).wait())


@jax.jit
def _sc_gather_multi(tabs, idx):
    dt = tabs[0].dtype
    f = pl.kernel(
        _gather_multi_body,
        out_type=[jax.ShapeDtypeStruct((NEP, HID), dt) for _ in tabs],
        mesh=_mesh(),
        scratch_types=[pltpu.VMEM((K, CH), jnp.int32),
                       pltpu.VMEM((2 * GG, CH, HID), dt),
                       pltpu.SemaphoreType.DMA,
                       pltpu.SemaphoreType.DMA,
                       pltpu.SemaphoreType.DMA,
                       pltpu.SemaphoreType.DMA],
        compiler_params=_SC_PARAMS,
    )
    return f(*tabs, idx)


def _sc_gather(tab, idx):
    return _sc_gather_multi((tab,), idx)[0]


def _scatter_body(msgx, obj_idx, zrows, out, idxb, rows, ls0, ls1, as0, as1,
                  shared):
    c = lax.axis_index("c")
    s = lax.axis_index("s")
    seg_per_tile = NSEG // NS
    t0 = s * seg_per_tile
    col0 = c * DOV

    pltpu.sync_copy(zrows, shared.at[pl.ds(t0, seg_per_tile)])
    plsc.subcore_barrier()

    def run_span(w):
        pltpu.sync_copy(obj_idx.at[w], idxb)
        base = w * (K * CH)
        lsem = (ls0, ls1)
        asem = (as0, as1)

        def lcp(j, slot, g):
            return pltpu.make_async_copy(
                msgx.at[pl.ds(base + j * CH, CH), pl.ds(col0, DMH)],
                rows.at[slot], lsem[g])

        def astart(j, slot, g):
            pltpu.async_copy(rows.at[slot], shared.at[idxb.at[j]], asem[g],
                             add=True)

        def await_(j, slot, g):
            pltpu.make_async_copy(rows.at[slot], shared.at[idxb.at[j]],
                                  asem[g]).wait()

        _dma_pipeline(K, GS, lcp, astart, await_)

    run_span(2 * s)
    run_span(2 * s + 1)
    plsc.subcore_barrier()
    pltpu.sync_copy(shared.at[pl.ds(t0, seg_per_tile)],
                    out.at[pl.ds(t0, seg_per_tile), pl.ds(col0, DMH)])


@jax.jit
def _sc_scatter(msgx, obj_idx, zrows):
    f = pl.kernel(
        _scatter_body,
        out_type=jax.ShapeDtypeStruct((NSEG, DM), jnp.float32),
        mesh=_mesh(),
        scratch_types=[pltpu.VMEM((K, CH), jnp.int32),
                       pltpu.VMEM((2 * GS, CH, DMH), jnp.float32),
                       pltpu.SemaphoreType.DMA,
                       pltpu.SemaphoreType.DMA,
                       pltpu.SemaphoreType.DMA,
                       pltpu.SemaphoreType.DMA,
                       pltpu.VMEM_SHARED((NSEG, DMH), jnp.float32)],
        compiler_params=_SC_PARAMS,
    )
    return f(msgx, obj_idx, zrows)


def _l0_body(m0s, key_idx, obj_idx, zrows, out, keyb, objb, rows,
             ls0, ls1, as0, as1, shared):
    c = lax.axis_index("c")
    s = lax.axis_index("s")
    seg_per_tile = NSEG // NS
    t0 = s * seg_per_tile

    pltpu.sync_copy(zrows, shared.at[pl.ds(t0, seg_per_tile)])
    plsc.subcore_barrier()

    m0c = m0s.at[c]

    def run_span(w):
        pltpu.sync_copy(key_idx.at[w], keyb)
        pltpu.sync_copy(obj_idx.at[w], objb)
        lsem = (ls0, ls1)
        asem = (as0, as1)

        def lcp(j, slot, g):
            return pltpu.make_async_copy(m0c.at[keyb.at[j]], rows.at[slot],
                                         lsem[g])

        def astart(j, slot, g):
            pltpu.async_copy(rows.at[slot], shared.at[objb.at[j]], asem[g],
                             add=True)

        def await_(j, slot, g):
            pltpu.make_async_copy(rows.at[slot], shared.at[objb.at[j]],
                                  asem[g]).wait()

        _dma_pipeline(K, GS, lcp, astart, await_)

    run_span(2 * s)
    run_span(2 * s + 1)
    plsc.subcore_barrier()
    pltpu.sync_copy(shared.at[pl.ds(t0, seg_per_tile)],
                    out.at[pl.ds(t0, seg_per_tile), pl.ds(c * DOV, DMH)])


@jax.jit
def _sc_l0(m0s, key_idx, obj_idx, zrows):
    f = pl.kernel(
        _l0_body,
        out_type=jax.ShapeDtypeStruct((NSEG, DM), jnp.float32),
        mesh=_mesh(),
        scratch_types=[pltpu.VMEM((K, CH), jnp.int32),
                       pltpu.VMEM((K, CH), jnp.int32),
                       pltpu.VMEM((2 * GS, CH, DMH), jnp.float32),
                       pltpu.SemaphoreType.DMA,
                       pltpu.SemaphoreType.DMA,
                       pltpu.SemaphoreType.DMA,
                       pltpu.SemaphoreType.DMA,
                       pltpu.VMEM_SHARED((NSEG, DMH), jnp.float32)],
        compiler_params=_SC_PARAMS,
    )
    return f(m0s, key_idx, obj_idx, zrows)


def _edge_body(hs_ref, hr_ref, r_ref, qre_ref, g1b_ref, g1m_ref, gb1_ref,
               g2b_ref, gb2_ref, Ws_ref, Wqr_ref, bqr_ref, wa_ref, ba_ref,
               out_ref):
    hsb = hs_ref[...]
    hrb = hr_ref[...]
    hs = hsb.astype(jnp.float32)
    nb = hs.shape[0]
    r = r_ref[0]
    iot = lax.broadcasted_iota(jnp.int32, (NQ, nb), 0)
    oh = (iot == r).astype(jnp.float32)
    qre = qre_ref[...]
    dn = (((0,), (0,)), ((), ()))
    f32 = jnp.float32
    g1b = g1b_ref[...]
    qg1 = qre @ g1m_ref[...]
    g = (lax.dot_general(hrb, g1b[:HID], (((1,), (0,)), ((), ())),
                         preferred_element_type=f32) +
         lax.dot_general(hsb, g1b[2 * HID:], (((1,), (0,)), ((), ())),
                         preferred_element_type=f32) +
         lax.dot_general(oh, qg1, dn, preferred_element_type=f32))
    g = 1.0 / (1.0 + jnp.exp(-(g + gb1_ref[...])))
    u = g[:, :HID]
    rs = g[:, HID:]
    g2b = g2b_ref[...]
    rh = (rs * hs).astype(jnp.bfloat16)
    cand = jnp.tanh(
        lax.dot_general(hrb, g2b[:HID], (((1,), (0,)), ((), ())),
                        preferred_element_type=f32) +
        lax.dot_general(rh, g2b[HID:], (((1,), (0,)), ((), ())),
                        preferred_element_type=f32) + gb2_ref[...])
    uri = (1.0 - u) * hs + u * cand
    qa = qre @ Wqr_ref[...] + bqr_ref[...]
    sc = jnp.maximum(uri @ Ws_ref[...] +
                     lax.dot_general(oh, qa, dn,
                                     preferred_element_type=f32), 0.0)
    aw = sc @ wa_ref[...] + ba_ref[...]
    ue = jnp.exp(aw)
    out_ref[...] = jnp.concatenate(
        [ue * uri, ue, jnp.zeros((nb, DM - HID - 1), jnp.float32)], axis=1)


def _full(shape):
    nd = len(shape)
    return pl.BlockSpec(shape, lambda i, _n=nd: (0,) * _n)


@jax.jit
def _tc_edge(hs, hr, r3, qre, g1b, g1m, gb1, g2b, gb2, Ws, Wqr, bqr, wa, ba):
    ne = hs.shape[0]
    nblk = ne // BE
    return pl.pallas_call(
        _edge_body,
        out_shape=jax.ShapeDtypeStruct((ne, DM), jnp.float32),
        grid=(nblk,),
        in_specs=[
            pl.BlockSpec((BE, HID), lambda i: (i, 0)),
            pl.BlockSpec((BE, HID), lambda i: (i, 0)),
            pl.BlockSpec((1, 1, BE), lambda i: (i, 0, 0)),
            _full((NQ, HID)), _full((3 * HID, 2 * HID)),
            _full((HID, 2 * HID)), _full((1, 2 * HID)),
            _full((2 * HID, HID)), _full((1, HID)), _full((HID, ATT)),
            _full((HID, ATT)), _full((1, ATT)), _full((ATT, 1)),
            _full((1, 1)),
        ],
        out_specs=pl.BlockSpec((BE, DM), lambda i: (i, 0)),
    )(hs, hr, r3, qre, g1b, g1m, gb1, g2b, gb2, Ws, Wqr, bqr, wa, ba)


def _node_body(a0_ref, ng_ref, h0_ref, qre_ref, Wh_ref, g1b_ref, g1m_ref,
               gb1_ref, g2b_ref, gb2_ref, Wf_ref, hid_ref, hbf_ref, sc_ref):
    a = a0_ref[...]
    agg = a[:, :HID]
    se = a[:, HID:HID + 1]
    m = jnp.maximum((agg / se) @ Wh_ref[...], 0.0)
    ng = ng_ref[0]
    iot = lax.broadcasted_iota(jnp.int32, (NQ, BN), 0)
    oh = (iot == ng).astype(jnp.float32)
    dn = (((0,), (0,)), ((), ()))
    f32 = jnp.float32
    h = h0_ref[...]
    g1 = g1b_ref[...]
    qg1 = qre_ref[...] @ g1m_ref[...]
    g = (m @ g1[:HID] + h @ g1[2 * HID:] +
         lax.dot_general(oh, qg1, dn, preferred_element_type=f32))
    g = 1.0 / (1.0 + jnp.exp(-(g + gb1_ref[...])))
    u = g[:, :HID]
    rs = g[:, HID:]
    g2 = g2b_ref[...]
    cand = jnp.tanh(m @ g2[:HID] + (rs * h) @ g2[HID:] + gb2_ref[...])
    out = (1.0 - u) * h + u * cand
    hid_ref[...] = out
    hbf_ref[...] = out.astype(jnp.bfloat16)
    sc_ref[...] = out @ Wf_ref[...]


@jax.jit
def _tc_node(a0, ng3, h0, qre, Wh, g1b, g1m, gb1, g2b, gb2, Wf):
    nblk = NNODE // BN
    return pl.pallas_call(
        _node_body,
        out_shape=[jax.ShapeDtypeStruct((NNODE, HID), jnp.float32),
                   jax.ShapeDtypeStruct((NNODE, HID), jnp.bfloat16),
                   jax.ShapeDtypeStruct((NNODE, 1), jnp.float32)],
        grid=(nblk,),
        in_specs=[
            pl.BlockSpec((BN, DM), lambda i: (i, 0)),
            pl.BlockSpec((1, 1, BN), lambda i: (i, 0, 0)),
            pl.BlockSpec((BN, HID), lambda i: (i, 0)),
            _full((NQ, HID)), _full((HID, HID)), _full((3 * HID, 2 * HID)),
            _full((HID, 2 * HID)), _full((1, 2 * HID)),
            _full((2 * HID, HID)), _full((1, HID)),
            _full((HID, 1)),
        ],
        out_specs=[pl.BlockSpec((BN, HID), lambda i: (i, 0)),
                   pl.BlockSpec((BN, HID), lambda i: (i, 0)),
                   pl.BlockSpec((BN, 1), lambda i: (i, 0))],
    )(a0, ng3, h0, qre, Wh, g1b, g1m, gb1, g2b, gb2, Wf)


def kernel(subs, rels, edges, nodes, old_idx, params):
    n = subs.shape[0]
    idt = edges.dtype
    r_idx = edges[:, 0]
    rel = edges[:, 2]
    sub = edges[:, 4]
    obj = edges[:, 5]
    sub2 = old_idx[sub]
    key = rel * NQ + r_idx

    node_group = jnp.zeros((NNODE,), dtype=idt).at[obj].set(r_idx)

    pad = NEP - NE

    def p32(x, fill):
        return jnp.concatenate(
            [x, jnp.full((pad,), fill, x.dtype)]).reshape(NW, K, CH)

    sub_sc = p32(sub, 0)
    sub2_sc = p32(sub2, 0)
    rel_sc = p32(rel, 0)
    obj_sc = p32(obj, NNODE)
    key_sc = p32(key, 0)
    r3 = jnp.concatenate([r_idx, jnp.zeros((pad,), idt)]).reshape(
        NEP // BE, 1, BE)
    ng3 = node_group.reshape(NNODE // BN, 1, BN)
    zrows = jnp.zeros((NSEG // NS, DMH), jnp.float32)

    layers = params["layers"]
    top = params["top"]
    bf = jnp.bfloat16
    tp = [top["gW1"], top["gW1"][HID:2 * HID],
          top["gb1"].reshape(1, -1), top["gW2"],
          top["gb2"].reshape(1, -1)]
    Wf = params["Wfinal"]

    def edge_call(p, hs, hr, r3v, qre):
        return _tc_edge(hs, hr, r3v, qre, p["gW1"].astype(bf),
                        p["gW1"][HID:2 * HID], p["gb1"].reshape(1, -1),
                        p["gW2"].astype(bf), p["gb2"].reshape(1, -1),
                        p["Ws"], p["Wqr"], p["bqr"].reshape(1, -1), p["wa"],
                        p["ba"].reshape(1, -1))

    def node_call(p, aggx, h0_in, qre):
        return _tc_node(aggx[:NNODE], ng3, h0_in, qre, p["Wh"],
                        tp[0], tp[1], tp[2], tp[3], tp[4], Wf)

    qres = [p["rela"][rels] for p in layers]

    hr_tabs = [layers[li]["rela"].astype(bf) for li in range(1, NL + NXL)]
    hr_outs = _sc_gather_multi(tuple(hr_tabs), rel_sc)
    hr_l = [None] + list(hr_outs)

    p0 = layers[0]
    hs0 = jnp.zeros((KEYP, HID), jnp.bfloat16)
    hr0 = jnp.concatenate(
        [jnp.repeat(p0["rela"], NQ, axis=0),
         jnp.zeros((KEYP - NKEY, HID), jnp.float32)]).astype(jnp.bfloat16)
    r0 = jnp.tile(jnp.arange(NQ, dtype=idt), KEYP // NQ).reshape(
        KEYP // BE, 1, BE)
    m0 = edge_call(p0, hs0, hr0, r0, qres[0])
    m0s = jnp.stack([m0[:, :DMH], m0[:, DOV:]])
    aggx = _sc_l0(m0s, key_sc, obj_sc, zrows)
    zeros_h = jnp.zeros((NNODE, HID), jnp.float32)
    hidden, hidden_bf, _ = node_call(p0, aggx, zeros_h, qres[0])
    h0 = hidden
    h0_in = jnp.zeros((NNODE, HID), jnp.float32).at[old_idx].set(h0)

    for li in range(1, NL + NXL):
        p = layers[li]
        sub_idx = sub_sc if li < NL else sub2_sc
        hs = _sc_gather(hidden_bf, sub_idx)
        msgx = edge_call(p, hs, hr_l[li], r3, qres[li])
        aggx = _sc_scatter(msgx, obj_sc, zrows)
        hidden, hidden_bf, sc = node_call(p, aggx, h0_in, qres[li])
        h0_in = hidden

    scores = sc[:, 0]
    scores_all = jnp.zeros((n, NNODE), jnp.float32).at[
        nodes[:, 0], nodes[:, 1]].set(scores)
    return scores_all
